# Initial kernel scaffold; baseline (speedup 1.0000x reference)
import jax
import jax.numpy as jnp
from jax.experimental import pallas as pl


def kernel(inputs):
    raise NotImplementedError("write your pallas kernel here")


# speedup vs baseline: 3.5291x; 3.5291x over previous
import functools
import math

import jax
import jax.numpy as jnp
from jax.experimental import pallas as pl

_NUM_UNITS = 1024
_SCALE = float(_NUM_UNITS) ** 0.5
_NEG2LOG1E4_OVER_D = -2.0 * math.log(10000.0) / _NUM_UNITS
_HALF_PI = math.pi / 2.0


def _pe_body(out_ref, *, block_t: int, batch: int):
    t0 = pl.program_id(0) * block_t
    shape = (block_t, _NUM_UNITS)
    pos_i = jax.lax.broadcasted_iota(jnp.int32, shape, 0) + t0
    ch_i = jax.lax.broadcasted_iota(jnp.int32, shape, 1)
    pos = pos_i.astype(jnp.float32)
    inv_freq = jnp.exp(ch_i.astype(jnp.float32) * _NEG2LOG1E4_OVER_D)
    is_odd = ch_i % 2
    angle = pos * inv_freq + is_odd.astype(jnp.float32) * _HALF_PI
    pe = jnp.sin(angle)
    pe = jnp.where(pos_i == 0, 0.0, pe) * _SCALE
    out_ref[...] = jnp.broadcast_to(pe[None], (batch,) + shape)


def kernel(inputs):
    n, t, d = inputs.shape
    block_t = 512
    body = functools.partial(_pe_body, block_t=block_t, batch=n)
    return pl.pallas_call(
        body,
        grid=(t // block_t,),
        out_shape=jax.ShapeDtypeStruct((n, t, d), jnp.float32),
        out_specs=pl.BlockSpec((n, block_t, d), lambda i: (0, i, 0)),
    )()


# speedup vs baseline: 7.3774x; 2.0905x over previous
import functools
import math

import jax
import jax.numpy as jnp
from jax.experimental import pallas as pl
from jax.experimental.pallas import tpu as pltpu

_NUM_UNITS = 1024
_SCALE = float(_NUM_UNITS) ** 0.5
_NEG2LOG1E4_OVER_D = -2.0 * math.log(10000.0) / _NUM_UNITS
_HALF_PI = math.pi / 2.0


def _pe_body(out_ref, sr_ref, cr_ref, *, block_t: int, batch: int):
    pid = pl.program_id(0)
    shape = (block_t, _NUM_UNITS)

    @pl.when(pid == 0)
    def _init_tables():
        r = jax.lax.broadcasted_iota(jnp.int32, shape, 0).astype(jnp.float32)
        ch = jax.lax.broadcasted_iota(jnp.int32, shape, 1).astype(jnp.float32)
        rw = r * jnp.exp(ch * _NEG2LOG1E4_OVER_D)
        sr_ref[...] = jnp.sin(rw)
        cr_ref[...] = jnp.sin(rw + _HALF_PI)

    ch1 = jax.lax.broadcasted_iota(jnp.int32, (1, _NUM_UNITS), 1)
    w1 = jnp.exp(ch1.astype(jnp.float32) * _NEG2LOG1E4_OVER_D)
    off = (ch1 % 2).astype(jnp.float32) * _HALF_PI
    phase = (pid * block_t).astype(jnp.float32) * w1 + off
    sb = jnp.sin(phase) * _SCALE
    cb = jnp.sin(phase + _HALF_PI) * _SCALE
    pe = sb * cr_ref[...] + cb * sr_ref[...]
    out_ref[...] = jnp.broadcast_to(pe[None], (batch,) + shape)

    @pl.when(pid == 0)
    def _zero_first_row():
        out_ref[:, 0:1, :] = jnp.zeros((batch, 1, _NUM_UNITS), jnp.float32)


def kernel(inputs):
    n, t, d = inputs.shape
    block_t = 512
    body = functools.partial(_pe_body, block_t=block_t, batch=n)
    return pl.pallas_call(
        body,
        grid=(t // block_t,),
        out_shape=jax.ShapeDtypeStruct((n, t, d), jnp.float32),
        out_specs=pl.BlockSpec((n, block_t, d), lambda i: (0, i, 0)),
        scratch_shapes=[
            pltpu.VMEM((block_t, d), jnp.float32),
            pltpu.VMEM((block_t, d), jnp.float32),
        ],
    )()


# speedup vs baseline: 8.5699x; 1.1616x over previous
import functools
import math

import jax
import jax.numpy as jnp
from jax.experimental import pallas as pl
from jax.experimental.pallas import tpu as pltpu

_NUM_UNITS = 1024
_SCALE = float(_NUM_UNITS) ** 0.5
_NEG2LOG1E4_OVER_D = -2.0 * math.log(10000.0) / _NUM_UNITS
_HALF_PI = math.pi / 2.0
_SUB = 64


def _pe_body(out_ref, s64_ref, c64_ref, *, block_t: int, batch: int):
    pid = pl.program_id(0)

    @pl.when(pid == 0)
    def _init_tables():
        r = jax.lax.broadcasted_iota(jnp.int32, (_SUB, _NUM_UNITS), 0)
        ch = jax.lax.broadcasted_iota(jnp.int32, (_SUB, _NUM_UNITS), 1)
        rw = r.astype(jnp.float32) * jnp.exp(
            ch.astype(jnp.float32) * _NEG2LOG1E4_OVER_D
        )
        s64_ref[...] = jnp.sin(rw)
        c64_ref[...] = jnp.sin(rw + _HALF_PI)

    ch1 = jax.lax.broadcasted_iota(jnp.int32, (1, _NUM_UNITS), 1)
    w1 = jnp.exp(ch1.astype(jnp.float32) * _NEG2LOG1E4_OVER_D)
    off = (ch1 % 2).astype(jnp.float32) * _HALF_PI
    s64 = s64_ref[...]
    c64 = c64_ref[...]
    for a in range(block_t // _SUB):
        p0f = (pid * block_t + a * _SUB).astype(jnp.float32)
        phase = p0f * w1 + off
        sb = jnp.sin(phase) * _SCALE
        cb = jnp.sin(phase + _HALF_PI) * _SCALE
        pe = sb * c64 + cb * s64
        out_ref[:, a * _SUB : (a + 1) * _SUB, :] = jnp.broadcast_to(
            pe[None], (batch, _SUB, _NUM_UNITS)
        )

    @pl.when(pid == 0)
    def _zero_first_row():
        out_ref[:, 0:1, :] = jnp.zeros((batch, 1, _NUM_UNITS), jnp.float32)


def kernel(inputs):
    n, t, d = inputs.shape
    block_t = 512
    body = functools.partial(_pe_body, block_t=block_t, batch=n)
    return pl.pallas_call(
        body,
        grid=(t // block_t,),
        out_shape=jax.ShapeDtypeStruct((n, t, d), jnp.float32),
        out_specs=pl.BlockSpec((n, block_t, d), lambda i: (0, i, 0)),
        scratch_shapes=[
            pltpu.VMEM((_SUB, d), jnp.float32),
            pltpu.VMEM((_SUB, d), jnp.float32),
        ],
    )()
